# baseline (speedup 1.0000x reference)
import functools

import jax
import jax.numpy as jnp
from jax import lax
from jax.experimental import pallas as pl
from jax.experimental.pallas import tpu as pltpu
from jax.experimental.pallas import tpu_sc as plsc

N_NODES = 10000
D = 128
N_EDGES = 320000

HALF = N_NODES // 2
NL = 5120
LOC_DUMMY = 5008
NW = 32
W_EDGES = 10240
EDGES_PAD = NW * W_EDGES
LIST_CAP = W_EDGES + 128
SCHUNK = 64
PIECE = 32
NPIECES = W_EDGES // (PIECE * SCHUNK)
RPT = NL // 16


def _edge_prep(src3g, dst3g):
    NG = W_EDGES // 16

    def body(s_ref, d_ref, pos_ref, pd_ref, tpos_ref, pcnt_ref):
        w = pl.program_id(0)
        dsp = d_ref[0]
        m0 = dsp < HALF
        m0f = m0.astype(jnp.float32)
        li = jnp.arange(16, dtype=jnp.int32)
        tril16 = (li[:, None] < li[None, :]).astype(jnp.float32)
        lane_excl = jnp.dot(m0f, tril16,
                            preferred_element_type=jnp.float32)
        g = jnp.sum(m0f, axis=1)
        gi = jnp.arange(NG, dtype=jnp.int32)
        tril = (gi[:, None] < gi[None, :]).astype(jnp.float32)
        gexcl = jnp.dot(g[None, :], tril,
                        preferred_element_type=jnp.float32)[0]
        tot0 = jnp.sum(g).astype(jnp.int32)

        pos0 = (gexcl[:, None] + lane_excl).astype(jnp.int32)
        gidx = jax.lax.broadcasted_iota(jnp.int32, (NG, 16), 0)
        lidx = jax.lax.broadcasted_iota(jnp.int32, (NG, 16), 1)
        idx_in_w = gidx * 16 + lidx
        pos1 = idx_in_w - pos0
        pos_ref[0] = jnp.where(m0, w * W_EDGES + pos0,
                               (NW + w) * W_EDGES + pos1)
        pd_ref[0] = jnp.where(m0, dsp, dsp - HALF)

        ti = jax.lax.broadcasted_iota(jnp.int32, (1, 64), 1)
        PARK = 2 * NW * W_EDGES
        p0 = tot0 + ti
        tail0 = jnp.where(p0 < W_EDGES, w * W_EDGES + p0, PARK)
        p1 = (W_EDGES - tot0) + ti
        tail1 = jnp.where(p1 < W_EDGES, (NW + w) * W_EDGES + p1, PARK)
        tpos_ref[...] = jnp.concatenate(
            [tail0, tail1], axis=1)[None]

        nch0 = (tot0 + SCHUNK - 1) // SCHUNK
        nch1 = (W_EDGES - tot0 + SCHUNK - 1) // SCHUNK
        nch = jnp.stack([nch0, nch1])
        pcnt_ref[...] = jnp.broadcast_to(
            nch[None, :, None, None], (1, 2, 8, 128))

    return pl.pallas_call(
        body,
        grid=(NW,),
        in_specs=[
            pl.BlockSpec((1, NG, 16), lambda i: (i, 0, 0)),
            pl.BlockSpec((1, NG, 16), lambda i: (i, 0, 0)),
        ],
        out_specs=(
            pl.BlockSpec((1, NG, 16), lambda i: (i, 0, 0)),
            pl.BlockSpec((1, NG, 16), lambda i: (i, 0, 0)),
            pl.BlockSpec((1, 1, 128), lambda i: (i, 0, 0)),
            pl.BlockSpec((1, 2, 8, 128), lambda i: (i, 0, 0, 0)),
        ),
        out_shape=(
            jax.ShapeDtypeStruct((NW, NG, 16), jnp.int32),
            jax.ShapeDtypeStruct((NW, NG, 16), jnp.int32),
            jax.ShapeDtypeStruct((NW, 1, 128), jnp.int32),
            jax.ShapeDtypeStruct((NW, 2, 8, 128), jnp.int32),
        ),
    )(src3g, dst3g)


def _partition_sc(pos3, ps3, pd3):
    mesh = plsc.VectorSubcoreMesh(core_axis_name="c", subcore_axis_name="s")

    @functools.partial(
        pl.kernel,
        out_type=(
            jax.ShapeDtypeStruct((2 * NW * W_EDGES + 128,), jnp.int32),
            jax.ShapeDtypeStruct((2 * NW * W_EDGES + 128,), jnp.int32),
        ),
        mesh=mesh,
        scratch_types=[
            pltpu.VMEM((81, 128), jnp.int32),
            pltpu.VMEM((81, 128), jnp.int32),
            pltpu.VMEM((81, 128), jnp.int32),
            [pltpu.SemaphoreType.DMA for _ in range(2)],
        ],
    )
    def k(pos_hbm, ps_hbm, pd_hbm, psrc_hbm, pdst_hbm,
          posv, psv, pdv, sems):
        c = lax.axis_index("c")
        s = lax.axis_index("s")
        w = s * 2 + c

        pltpu.sync_copy(pos_hbm.at[w], posv)
        pltpu.sync_copy(ps_hbm.at[w], psv)
        pltpu.sync_copy(pd_hbm.at[w], pdv)

        def ci_body(ci, _):
            pltpu.async_copy(psv.at[ci], psrc_hbm.at[posv.at[ci]], sems[0])
            pltpu.async_copy(pdv.at[ci], pdst_hbm.at[posv.at[ci]], sems[1])
            pltpu.make_async_copy(
                psv.at[ci], psrc_hbm.at[posv.at[ci]], sems[0]).wait()
            pltpu.make_async_copy(
                pdv.at[ci], pdst_hbm.at[posv.at[ci]], sems[1]).wait()
            return 0

        lax.fori_loop(0, 81, ci_body, 0)

    return k(pos3, ps3, pd3)


def _segment_sum_sc(hw, psrc4, pdst4, pcnt, zeros):
    mesh = plsc.VectorSubcoreMesh(core_axis_name="c", subcore_axis_name="s")

    @functools.partial(
        pl.kernel,
        out_type=jax.ShapeDtypeStruct((2, NL, D), jnp.float32),
        mesh=mesh,
        scratch_types=[
            pltpu.VMEM((PIECE, SCHUNK), jnp.int32),
            pltpu.VMEM((PIECE, SCHUNK), jnp.int32),
            [pltpu.VMEM((SCHUNK, D), jnp.float32) for _ in range(2)],
            pltpu.VMEM((8, 128), jnp.int32),
            pltpu.VMEM_SHARED((NL, D), jnp.float32),
            [pltpu.SemaphoreType.DMA for _ in range(2)],
            [pltpu.SemaphoreType.DMA for _ in range(2)],
        ],
    )
    def k(hw_hbm, psrc_hbm, pdst_hbm, pcnt_hbm, zero_hbm, out_hbm,
          sv, dvv, bufs, cntv, acc, gsems, ssems):
        c = lax.axis_index("c")
        s = lax.axis_index("s")

        pltpu.sync_copy(zero_hbm.at[pl.ds(s * RPT, RPT)],
                        acc.at[pl.ds(s * RPT, RPT)])
        plsc.subcore_barrier()

        for w2 in range(2):
            w = 2 * s + w2
            L = c * NW + w
            pltpu.sync_copy(pcnt_hbm.at[w, c], cntv)
            nch = cntv[0, pl.ds(0, 16)][0]
            npieces = (nch + PIECE - 1) // PIECE

            def piece_body(pi, _):
                pltpu.sync_copy(psrc_hbm.at[L, pi], sv)
                pltpu.sync_copy(pdst_hbm.at[L, pi], dvv)
                k_hi = jnp.minimum(nch - pi * PIECE, PIECE)

                pltpu.async_copy(hw_hbm.at[sv.at[0]], bufs[0], gsems[0])

                @pl.when(k_hi > 1)
                def _():
                    pltpu.async_copy(hw_hbm.at[sv.at[1]], bufs[1], gsems[1])

                def pair(i, _):
                    j0 = i * 2
                    for b in range(2):
                        j = j0 + b
                        pltpu.make_async_copy(
                            hw_hbm.at[sv.at[j]], bufs[b], gsems[b]).wait()
                        pltpu.async_copy(
                            bufs[b], acc.at[dvv.at[j]], ssems[b], add=True)
                        pltpu.make_async_copy(
                            bufs[b], acc.at[dvv.at[j]], ssems[b]).wait()

                        @pl.when(j + 2 < k_hi)
                        def _():
                            pltpu.async_copy(
                                hw_hbm.at[sv.at[j + 2]], bufs[b], gsems[b])
                    return 0

                lax.fori_loop(0, k_hi // 2, pair, 0)

                @pl.when(k_hi % 2 == 1)
                def _():
                    j = k_hi - 1
                    pltpu.make_async_copy(
                        hw_hbm.at[sv.at[j]], bufs[0], gsems[0]).wait()
                    pltpu.async_copy(
                        bufs[0], acc.at[dvv.at[j]], ssems[0], add=True)
                    pltpu.make_async_copy(
                        bufs[0], acc.at[dvv.at[j]], ssems[0]).wait()

                return 0

            lax.fori_loop(0, npieces, piece_body, 0)

        plsc.subcore_barrier()
        pltpu.sync_copy(acc.at[pl.ds(s * RPT, RPT)],
                        out_hbm.at[c, pl.ds(s * RPT, RPT)])

    return k(hw, psrc4, pdst4, pcnt, zeros)


def _mm_first(x, W):
    def body(x_ref, w_ref, o_ref):
        o_ref[...] = jnp.dot(x_ref[...], w_ref[...],
                             preferred_element_type=jnp.float32)

    return pl.pallas_call(
        body,
        out_shape=jax.ShapeDtypeStruct((N_NODES, D), jnp.float32),
    )(x, W)


def _bn_relu(p_ref, g_ref, b_ref):
    sarr = jnp.concatenate(
        [p_ref[0, pl.ds(0, HALF), :], p_ref[1, pl.ds(0, HALF), :]], axis=0)
    mu = jnp.mean(sarr, axis=0, keepdims=True)
    d = sarr - mu
    var = jnp.mean(d * d, axis=0, keepdims=True)
    hn = g_ref[...] * d * lax.rsqrt(var + 1e-5) + b_ref[...]
    return jnp.maximum(hn, 0.0)


def _stage_mid(p, g, b, W):
    def body(p_ref, g_ref, b_ref, w_ref, o_ref):
        h = _bn_relu(p_ref, g_ref, b_ref)
        o_ref[...] = jnp.dot(h, w_ref[...],
                             preferred_element_type=jnp.float32)

    return pl.pallas_call(
        body,
        out_shape=jax.ShapeDtypeStruct((N_NODES, D), jnp.float32),
    )(p, g.reshape(1, D), b.reshape(1, D), W)


def _stage_final(p, g, b, Wm0, bm0, Wm1, bm1):
    def body(p_ref, g_ref, b_ref, w0_ref, b0_ref, w1_ref, b1_ref, o_ref):
        h = _bn_relu(p_ref, g_ref, b_ref)
        pooled = jnp.sum(h, axis=0, keepdims=True)
        z = jnp.maximum(
            jnp.dot(pooled, w0_ref[...],
                    preferred_element_type=jnp.float32) + b0_ref[...], 0.0)
        o_ref[...] = jnp.dot(z, w1_ref[...],
                             preferred_element_type=jnp.float32) + b1_ref[...]

    return pl.pallas_call(
        body,
        out_shape=jax.ShapeDtypeStruct((1, 16), jnp.float32),
    )(p, g.reshape(1, D), b.reshape(1, D),
      Wm0, bm0.reshape(1, -1), Wm1, bm1.reshape(1, -1))


@jax.jit
def kernel(x, edge_index, W0, g0, b0, W1, g1, b1, W2, g2, b2,
           Wm0, bm0, Wm1, bm1):
    pad = EDGES_PAD - N_EDGES
    src3g = jnp.concatenate(
        [edge_index[0].astype(jnp.int32), jnp.zeros((pad,), jnp.int32)]
    ).reshape(NW, W_EDGES // 16, 16)
    dst3g = jnp.concatenate(
        [edge_index[1].astype(jnp.int32),
         jnp.full((pad,), HALF + LOC_DUMMY, jnp.int32)]
    ).reshape(NW, W_EDGES // 16, 16)
    posw, pdw, tpos, pcnt = _edge_prep(src3g, dst3g)
    pos3 = jnp.concatenate(
        [posw.reshape(NW, W_EDGES), tpos.reshape(NW, 128)],
        axis=1).reshape(NW, 81, 128)
    ps3 = jnp.concatenate(
        [src3g.reshape(NW, W_EDGES),
         jnp.zeros((NW, 128), jnp.int32)], axis=1).reshape(NW, 81, 128)
    pd3 = jnp.concatenate(
        [pdw.reshape(NW, W_EDGES),
         jnp.full((NW, 128), LOC_DUMMY, jnp.int32)], axis=1
    ).reshape(NW, 81, 128)
    psrc, pdst = _partition_sc(pos3, ps3, pd3)
    psrc4 = psrc[:2 * NW * W_EDGES].reshape(2 * NW, NPIECES, PIECE, SCHUNK)
    pdst4 = pdst[:2 * NW * W_EDGES].reshape(2 * NW, NPIECES, PIECE, SCHUNK)
    zeros = jnp.zeros((NL, D), jnp.float32)

    hw = _mm_first(x, W0)
    p = _segment_sum_sc(hw, psrc4, pdst4, pcnt, zeros)
    hw = _stage_mid(p, g0, b0, W1)
    p = _segment_sum_sc(hw, psrc4, pdst4, pcnt, zeros)
    hw = _stage_mid(p, g1, b1, W2)
    p = _segment_sum_sc(hw, psrc4, pdst4, pcnt, zeros)
    return _stage_final(p, g2, b2, Wm0, bm0, Wm1, bm1)


# speedup vs baseline: 1.8723x; 1.8723x over previous
import functools

import jax
import jax.numpy as jnp
from jax import lax
from jax.experimental import pallas as pl
from jax.experimental.pallas import tpu as pltpu
from jax.experimental.pallas import tpu_sc as plsc

N_NODES = 10000
D = 128
N_EDGES = 320000

NT = 16
CHUNK = 128
NCHUNK = 160
EDGES_PAD = NT * NCHUNK * CHUNK
NB = 2
PASSES = 4
PCHUNK = NCHUNK // PASSES
N_PAD = 10240
DUMMY_ROW = 10016
ROWS_PER_TILE = N_PAD // NT


def _segment_sum_sc(hw, src3, dst3, zeros):
    mesh = plsc.VectorSubcoreMesh(
        core_axis_name="c", subcore_axis_name="s", num_cores=1)

    @functools.partial(
        pl.kernel,
        out_type=jax.ShapeDtypeStruct((N_PAD, D), jnp.float32),
        mesh=mesh,
        scratch_types=[
            pltpu.VMEM((PCHUNK, CHUNK), jnp.int32),
            pltpu.VMEM((PCHUNK, CHUNK), jnp.int32),
            [pltpu.VMEM((CHUNK, D), jnp.float32) for _ in range(NB)],
            pltpu.VMEM_SHARED((N_PAD, D), jnp.float32),
            [pltpu.SemaphoreType.DMA for _ in range(NB)],
            [pltpu.SemaphoreType.DMA for _ in range(NB)],
        ],
    )
    def k(hw_hbm, src_hbm, dst_hbm, zero_hbm, out_hbm,
          src_v, dst_v, bufs, acc, gsems, ssems):
        s = lax.axis_index("s")

        pltpu.sync_copy(
            zero_hbm.at[pl.ds(s * ROWS_PER_TILE, ROWS_PER_TILE)],
            acc.at[pl.ds(s * ROWS_PER_TILE, ROWS_PER_TILE)])
        plsc.subcore_barrier()

        for p in range(PASSES):
            pltpu.sync_copy(src_hbm.at[s, pl.ds(p * PCHUNK, PCHUNK)], src_v)
            pltpu.sync_copy(dst_hbm.at[s, pl.ds(p * PCHUNK, PCHUNK)], dst_v)

            for b in range(NB):
                pltpu.async_copy(hw_hbm.at[src_v.at[b]], bufs[b], gsems[b])

            def body(i, _):
                j0 = i * NB
                for b in range(NB):
                    j = j0 + b
                    pltpu.make_async_copy(
                        hw_hbm.at[src_v.at[j]], bufs[b], gsems[b]).wait()
                    pltpu.async_copy(
                        bufs[b], acc.at[dst_v.at[j]], ssems[b], add=True)
                    pltpu.make_async_copy(
                        bufs[b], acc.at[dst_v.at[j]], ssems[b]).wait()

                    @pl.when(j + NB < PCHUNK)
                    def _():
                        pltpu.async_copy(
                            hw_hbm.at[src_v.at[j + NB]], bufs[b], gsems[b])

                return 0

            lax.fori_loop(0, PCHUNK // NB, body, 0)

        plsc.subcore_barrier()
        pltpu.sync_copy(
            acc.at[pl.ds(s * ROWS_PER_TILE, ROWS_PER_TILE)],
            out_hbm.at[pl.ds(s * ROWS_PER_TILE, ROWS_PER_TILE)],
        )

    return k(hw, src3, dst3, zeros)


def _mm_first(x, W):
    def body(x_ref, w_ref, o_ref):
        o_ref[...] = jnp.dot(x_ref[...], w_ref[...],
                             preferred_element_type=jnp.float32)

    return pl.pallas_call(
        body,
        out_shape=jax.ShapeDtypeStruct((N_NODES, D), jnp.float32),
    )(x, W)


def _bn_relu(p_ref, g_ref, b_ref):
    sarr = p_ref[pl.ds(0, N_NODES), :]
    mu = jnp.mean(sarr, axis=0, keepdims=True)
    d = sarr - mu
    var = jnp.mean(d * d, axis=0, keepdims=True)
    hn = g_ref[...] * d * lax.rsqrt(var + 1e-5) + b_ref[...]
    return jnp.maximum(hn, 0.0)


def _stage_mid(p, g, b, W):
    def body(p_ref, g_ref, b_ref, w_ref, o_ref):
        h = _bn_relu(p_ref, g_ref, b_ref)
        o_ref[...] = jnp.dot(h, w_ref[...],
                             preferred_element_type=jnp.float32)

    return pl.pallas_call(
        body,
        out_shape=jax.ShapeDtypeStruct((N_NODES, D), jnp.float32),
    )(p, g.reshape(1, D), b.reshape(1, D), W)


def _stage_final(p, g, b, Wm0, bm0, Wm1, bm1):
    def body(p_ref, g_ref, b_ref, w0_ref, b0_ref, w1_ref, b1_ref,
             o_ref):
        h = _bn_relu(p_ref, g_ref, b_ref)
        pooled = jnp.sum(h, axis=0, keepdims=True)
        z = jnp.maximum(
            jnp.dot(pooled, w0_ref[...],
                    preferred_element_type=jnp.float32) + b0_ref[...], 0.0)
        o_ref[...] = jnp.dot(z, w1_ref[...],
                             preferred_element_type=jnp.float32) + b1_ref[...]

    return pl.pallas_call(
        body,
        out_shape=jax.ShapeDtypeStruct((1, 16), jnp.float32),
    )(p, g.reshape(1, D), b.reshape(1, D),
      Wm0, bm0.reshape(1, -1), Wm1, bm1.reshape(1, -1))


@jax.jit
def kernel(x, edge_index, W0, g0, b0, W1, g1, b1, W2, g2, b2,
           Wm0, bm0, Wm1, bm1):
    pad = EDGES_PAD - N_EDGES
    src3 = jnp.concatenate(
        [edge_index[0].astype(jnp.int32), jnp.zeros((pad,), jnp.int32)]
    ).reshape(NT, NCHUNK, CHUNK)
    dst3 = jnp.concatenate(
        [edge_index[1].astype(jnp.int32),
         jnp.full((pad,), DUMMY_ROW, jnp.int32)]
    ).reshape(NT, NCHUNK, CHUNK)
    zeros = jnp.zeros((N_PAD, D), jnp.float32)

    hw = _mm_first(x, W0)
    p = _segment_sum_sc(hw, src3, dst3, zeros)
    hw = _stage_mid(p, g0, b0, W1)
    p = _segment_sum_sc(hw, src3, dst3, zeros)
    hw = _stage_mid(p, g1, b1, W2)
    p = _segment_sum_sc(hw, src3, dst3, zeros)
    return _stage_final(p, g2, b2, Wm0, bm0, Wm1, bm1)
